# baseline (speedup 1.0000x reference)
import functools

import jax
import jax.numpy as jnp
from jax import lax
from jax.experimental import pallas as pl
from jax.experimental.pallas import tpu as pltpu
from jax.experimental.pallas import tpu_sc as plsc

_C = 104
_CP = 112


@functools.partial(jax.jit, static_argnums=2)
def _gather3(table2, idxp, out_shape):
    R, TW = table2.shape
    D = TW // 2
    I, J, _ = out_shape
    info = plsc.get_sparse_core_info()
    NC = info.num_cores
    NW = NC * info.num_subcores
    ipw = I // NW
    n = 2 * ipw
    assert I == NW * ipw and n % 4 == 0 and idxp.shape == (I * 2 * _C,)

    mesh = plsc.VectorSubcoreMesh(core_axis_name="c", subcore_axis_name="s")

    @functools.partial(
        pl.kernel,
        mesh=mesh,
        out_type=jax.ShapeDtypeStruct((I, J, D), jnp.float32),
        scratch_types=[
            [pltpu.VMEM((_CP,), jnp.int32)] * 4,
            [pltpu.VMEM((_CP,), jnp.int32)] * 2,
            [pltpu.VMEM((_CP,), jnp.int32)] * 2,
            [pltpu.VMEM((_CP, TW), jnp.float32)] * 2,
            [pltpu.VMEM((_C, D), jnp.float32)] * 2,
            [pltpu.SemaphoreType.DMA] * 4,
            [pltpu.SemaphoreType.DMA] * 2,
            [pltpu.SemaphoreType.DMA] * 2,
        ],
        compiler_params=pltpu.CompilerParams(
            use_tc_tiling_on_sc=True, needs_layout_passes=False),
    )
    def k(tbl, idxs, out, idx_v, i2, par, rows, dsel, is_, gs, ws):
        wid = lax.axis_index("s") * NC + lax.axis_index("c")
        i0 = wid * ipw
        iota16 = lax.iota(jnp.int32, 16)
        tmask = iota16 < (_C - 6 * 16)

        def idesc(g, s):
            off = (i0 + lax.shift_right_logical(g, 1)) * 2 * _C \
                + lax.rem(g, 2) * _C
            return pltpu.make_async_copy(
                idxs.at[pl.ds(off, _C)], idx_v[s].at[pl.ds(0, _C)], is_[s])

        def gdesc(p):
            return pltpu.make_async_copy(tbl.at[i2[p]], rows[p], gs[p])

        def wdesc(g, p):
            i = i0 + lax.shift_right_logical(g, 1)
            j0, w = (0, _C) if p == 0 else (_C, J - _C)
            return pltpu.make_async_copy(
                dsel[p].at[pl.ds(0, w)], out.at[i, pl.ds(j0, w), :], ws[p])

        def prep_and_fire(g, s, p):
            idesc(g, s).wait()
            for kb in range(_CP // 16):
                v = idx_v[s][pl.ds(kb * 16, 16)]
                if (kb + 1) * 16 > _C:
                    v = jnp.where(tmask, v, 0)
                i2[p][pl.ds(kb * 16, 16)] = jnp.right_shift(v, 1)
                par[p][pl.ds(kb * 16, 16)] = jnp.bitwise_and(v, 1) * D
            gdesc(p).start()

        def select(p):
            for rb in range(_CP // 16):
                rid = iota16 + (rb * 16)
                p64 = par[p][pl.ds(rb * 16, 16)]
                partial = (rb + 1) * 16 > _C

                def col_block(cb, carry):
                    c0 = cb * 16
                    for kk in range(16):
                        cc = jnp.bitwise_and(iota16 + kk, 15) + c0
                        if partial:
                            vals = plsc.load_gather(
                                rows[p], [rid, cc + p64], mask=tmask)
                            plsc.store_scatter(
                                dsel[p], [rid, cc], vals, mask=tmask)
                        else:
                            vals = plsc.load_gather(rows[p], [rid, cc + p64])
                            plsc.store_scatter(dsel[p], [rid, cc], vals)
                    return carry

                lax.fori_loop(0, D // 16, col_block, 0, unroll=False)

        for g in range(4):
            idesc(g, g).start()
        prep_and_fire(0, 0, 0)
        prep_and_fire(1, 1, 1)

        def body(g2, carry):
            ge = lax.rem(g2, 2)
            for p in (0, 1):
                g = 2 * g2 + p
                gdesc(p).wait()

                @pl.when(g >= 2)
                def _():
                    wdesc(g - 2, p).wait()

                select(p)
                wdesc(g, p).start()

                @pl.when(g + 4 < n)
                def _():
                    pl.when(ge == 0)(lambda: idesc(g + 4, p).start())
                    pl.when(ge == 1)(lambda: idesc(g + 4, p + 2).start())

                @pl.when(g + 2 < n)
                def _():
                    pl.when(ge == 0)(lambda: prep_and_fire(g + 2, p + 2, p))
                    pl.when(ge == 1)(lambda: prep_and_fire(g + 2, p, p))

            return carry

        lax.fori_loop(0, n // 2, body, 0, unroll=False)
        wdesc(n - 1, 1).wait()
        wdesc(n - 2, 0).wait()

    return k(table2, idxp)


def kernel(input_, indices):
    V, D = input_.shape
    I, J = indices.shape
    table2 = input_.reshape(V // 2, 2 * D)
    idxp = jnp.pad(indices, ((0, 0), (0, 2 * _C - J))).reshape(I * 2 * _C)
    return _gather3(table2, idxp, (I, J, D))


# speedup vs baseline: 3.8172x; 3.8172x over previous
import functools

import jax
import jax.numpy as jnp
from jax import lax
from jax.experimental import pallas as pl
from jax.experimental.pallas import tpu as pltpu
from jax.experimental.pallas import tpu_sc as plsc

_C = 64


@jax.jit
def _gather_flat(table2, idx):
    R, TW = table2.shape
    D = TW // 2
    (B,) = idx.shape
    info = plsc.get_sparse_core_info()
    NC = info.num_cores
    NW = NC * info.num_subcores
    b_per_w = B // NW
    assert B == NW * b_per_w and b_per_w % (4 * _C) == 0
    n = b_per_w // _C

    mesh = plsc.VectorSubcoreMesh(core_axis_name="c", subcore_axis_name="s")

    @functools.partial(
        pl.kernel,
        mesh=mesh,
        out_type=jax.ShapeDtypeStruct((B, D), jnp.float32),
        scratch_types=[
            [pltpu.VMEM((_C,), jnp.int32)] * 4,
            [pltpu.VMEM((_C,), jnp.int32)] * 2,
            [pltpu.VMEM((_C,), jnp.int32)] * 2,
            [pltpu.VMEM((_C, TW), jnp.float32)] * 2,
            [pltpu.VMEM((_C, D), jnp.float32)] * 2,
            [pltpu.SemaphoreType.DMA] * 4,
            [pltpu.SemaphoreType.DMA] * 2,
            [pltpu.SemaphoreType.DMA] * 2,
        ],
        compiler_params=pltpu.CompilerParams(
            use_tc_tiling_on_sc=True, needs_layout_passes=False),
    )
    def k(tbl, idxs, out, idx_v, i2, par, rows, dsel, is_, gs, ws):
        wid = lax.axis_index("s") * NC + lax.axis_index("c")
        base = wid * b_per_w
        iota16 = lax.iota(jnp.int32, 16)

        def idesc(j, s):
            return pltpu.make_async_copy(
                idxs.at[pl.ds(base + j * _C, _C)], idx_v[s], is_[s])

        def gdesc(p):
            return pltpu.make_async_copy(tbl.at[i2[p]], rows[p], gs[p])

        def wdesc(j, p):
            return pltpu.make_async_copy(
                dsel[p], out.at[pl.ds(base + j * _C, _C)], ws[p])

        def prep_and_fire(j, s, p):
            idesc(j, s).wait()
            for kb in range(_C // 16):
                v = idx_v[s][pl.ds(kb * 16, 16)]
                i2[p][pl.ds(kb * 16, 16)] = jnp.right_shift(v, 1)
                par[p][pl.ds(kb * 16, 16)] = jnp.bitwise_and(v, 1) * D
            gdesc(p).start()

        def select(p):
            for rb in range(_C // 16):
                rid = iota16 + (rb * 16)
                p64 = par[p][pl.ds(rb * 16, 16)]
                for cb in range(D // 16):
                    c0 = cb * 16
                    for kk in range(16):
                        cc = jnp.bitwise_and(iota16 + kk, 15) + c0
                        vals = plsc.load_gather(rows[p], [rid, cc + p64])
                        plsc.store_scatter(dsel[p], [rid, cc], vals)

        for j in range(4):
            idesc(j, j).start()
        prep_and_fire(0, 0, 0)
        prep_and_fire(1, 1, 1)

        def body(g, carry):
            ge = lax.rem(g, 2)
            for p in (0, 1):
                j = 2 * g + p
                gdesc(p).wait()

                @pl.when(j >= 2)
                def _():
                    wdesc(j - 2, p).wait()

                select(p)
                wdesc(j, p).start()

                @pl.when(j + 4 < n)
                def _():
                    pl.when(ge == 0)(lambda: idesc(j + 4, p).start())
                    pl.when(ge == 1)(lambda: idesc(j + 4, p + 2).start())

                @pl.when(j + 2 < n)
                def _():
                    pl.when(ge == 0)(lambda: prep_and_fire(j + 2, p + 2, p))
                    pl.when(ge == 1)(lambda: prep_and_fire(j + 2, p, p))

            return carry

        lax.fori_loop(0, n // 2, body, 0, unroll=False)
        wdesc(n - 1, 1).wait()
        wdesc(n - 2, 0).wait()

    return k(table2, idx)


def kernel(input_, indices):
    V, D = input_.shape
    I, J = indices.shape
    table2 = input_.reshape(V // 2, 2 * D)
    flat = _gather_flat(table2, indices.reshape(I * J))
    return flat.reshape(I, J, D)
